# baseline (speedup 1.0000x reference)
import jax
import jax.numpy as jnp
from jax import lax
from jax.experimental import pallas as pl
from jax.experimental.pallas import tpu as pltpu
from jax.experimental.pallas import tpu_sc as plsc

_NC, _NS = 2, 16
_NW = _NC * _NS


def _dot0(a, b):
    return lax.dot_general(a, b, (((0,), (0,)), ((), ())),
                           preferred_element_type=jnp.float32)


def _prep_body(ttT_ref, wt_ref, wkt_ref, wwk_ref, ntT_ref, wn_ref,
               tp_ref, wp_ref, s_ref):
    tp_ref[:] = _dot0(ttT_ref[:], wt_ref[:])
    wproj = jnp.dot(wkt_ref[:], wwk_ref[:],
                    preferred_element_type=jnp.float32)
    wp_ref[:] = jnp.concatenate(
        [wproj, jnp.zeros((1, wproj.shape[1]), jnp.float32)], axis=0)
    s_ref[:] = _dot0(ntT_ref[:], wn_ref[:])


def _sc_gather_body(per, BT):
    def body(idx_hbm, tp_hbm, wp_hbm, out_hbm, ti_v, wi_v, rt_v, rw_v,
             s1, s2):
        wid = lax.axis_index("s") * _NC + lax.axis_index("c")
        base = wid * per
        pltpu.sync_copy(idx_hbm.at[pl.ds(base, per)], ti_v)
        pltpu.sync_copy(idx_hbm.at[pl.ds(BT + base, per)], wi_v)
        c1 = pltpu.async_copy(tp_hbm.at[ti_v], rt_v, s1)
        c2 = pltpu.async_copy(wp_hbm.at[wi_v], rw_v, s2)
        c1.wait()
        c2.wait()
        pltpu.sync_copy(rt_v, out_hbm.at[pl.ds(base, per)])
        pltpu.sync_copy(rw_v, out_hbm.at[pl.ds(BT + base, per)])
    return body


def _tc_body(g_ref, s_ref, out_ref):
    t = pl.program_id(0)
    N = out_ref.shape[1]
    B = out_ref.shape[2]
    BT = g_ref.shape[0] // 2

    tv = g_ref[pl.ds(t * B, B), :] + g_ref[pl.ds(BT + t * B, B), :]
    for n in range(N):
        out_ref[0, n] = tv + s_ref[n, :]


def kernel(time, weekday, time_table, W_time, weekday_table, W_weekday,
           node_table, W_node):
    B, T, _ = time.shape
    N, E = node_table.shape
    D = W_node.shape[1]
    BT = B * T
    per = BT // _NW
    Vt = time_table.shape[0]

    idx = jnp.concatenate(
        [time.reshape(B, T).T, weekday.reshape(B, T).T],
        axis=0).astype(jnp.int32).reshape(2 * BT)

    def full(shape):
        return pl.BlockSpec(shape, lambda *_: (0,) * len(shape))

    tproj, wproj, s = pl.pallas_call(
        _prep_body,
        in_specs=[full((E, Vt)), full(W_time.shape),
                  full(weekday_table.shape), full(W_weekday.shape),
                  full((E, N)), full(W_node.shape)],
        out_specs=[full((Vt, D)), full((8, D)), full((N, D))],
        out_shape=[jax.ShapeDtypeStruct((Vt, D), jnp.float32),
                   jax.ShapeDtypeStruct((8, D), jnp.float32),
                   jax.ShapeDtypeStruct((N, D), jnp.float32)],
    )(time_table.T, W_time, weekday_table, W_weekday, node_table.T, W_node)

    mesh = plsc.VectorSubcoreMesh(core_axis_name="c", subcore_axis_name="s")
    gathered = pl.kernel(
        _sc_gather_body(per, BT),
        out_type=jax.ShapeDtypeStruct((2 * BT, D), jnp.float32),
        mesh=mesh,
        scratch_types=[
            pltpu.VMEM((per,), jnp.int32),
            pltpu.VMEM((per,), jnp.int32),
            pltpu.VMEM((per, D), jnp.float32),
            pltpu.VMEM((per, D), jnp.float32),
            pltpu.SemaphoreType.DMA,
            pltpu.SemaphoreType.DMA,
        ],
    )(idx, tproj, wproj)

    out = pl.pallas_call(
        _tc_body,
        grid=(T,),
        in_specs=[full((2 * BT, D)), full((N, D))],
        out_specs=pl.BlockSpec((1, N, B, D), lambda t: (t, 0, 0, 0)),
        out_shape=jax.ShapeDtypeStruct((T, N, B, D), jnp.float32),
        compiler_params=pltpu.CompilerParams(
            dimension_semantics=("arbitrary",)),
    )(gathered, s)
    return jnp.transpose(out, (2, 0, 1, 3))


# speedup vs baseline: 1.4460x; 1.4460x over previous
import jax
import jax.numpy as jnp
from jax import lax
from jax.experimental import pallas as pl
from jax.experimental.pallas import tpu as pltpu


def _dot0(a, b):
    return lax.dot_general(a, b, (((0,), (0,)), ((), ())),
                           preferred_element_type=jnp.float32)


def _body(idx_ref, ttT_ref, wt_ref, wkt_ref, wwk_ref, ntT_ref, wn_ref,
          out_ref, s_scr):
    t = pl.program_id(0)
    j = pl.program_id(1)
    T = pl.num_programs(0)
    NB = out_ref.shape[1]
    B = out_ref.shape[2]
    Vt = ttT_ref.shape[1]

    @pl.when(jnp.logical_and(t == 0, j == 0))
    def _():
        s_scr[:] = _dot0(ntT_ref[:], wn_ref[:])

    ti = idx_ref[t]
    wi = idx_ref[T + t]
    oh_t = (ti[None, :] == lax.broadcasted_iota(jnp.int32, (Vt, B), 0)
            ).astype(jnp.float32)
    oh_w = (wi[None, :] == lax.broadcasted_iota(jnp.int32, (8, B), 0)
            ).astype(jnp.float32)
    tproj = _dot0(ttT_ref[:], wt_ref[:])
    wproj = jnp.dot(wkt_ref[:], wwk_ref[:],
                    preferred_element_type=jnp.float32)
    wproj8 = jnp.concatenate(
        [wproj, jnp.zeros((1, wproj.shape[1]), jnp.float32)], axis=0)
    tv = _dot0(oh_t, tproj) + _dot0(oh_w, wproj8)

    for n in range(NB):
        out_ref[0, n] = tv + s_scr[j * NB + n, :]


def kernel(time, weekday, time_table, W_time, weekday_table, W_weekday,
           node_table, W_node):
    B, T, _ = time.shape
    N, _ = node_table.shape
    D = W_node.shape[1]
    idx = jnp.concatenate(
        [time.reshape(B, T).T, weekday.reshape(B, T).T],
        axis=0).astype(jnp.int32)

    NJ = 3
    NB = N // NJ

    def full(shape):
        return pl.BlockSpec(shape, lambda *_: (0,) * len(shape))

    out = pl.pallas_call(
        _body,
        grid=(T, NJ),
        in_specs=[full(idx.shape),
                  full((time_table.shape[1], time_table.shape[0])),
                  full(W_time.shape), full(weekday_table.shape),
                  full(W_weekday.shape),
                  full((node_table.shape[1], node_table.shape[0])),
                  full(W_node.shape)],
        out_specs=pl.BlockSpec((1, NB, B, D), lambda t, j: (t, j, 0, 0)),
        out_shape=jax.ShapeDtypeStruct((T, N, B, D), jnp.float32),
        scratch_shapes=[pltpu.VMEM((N, D), jnp.float32)],
        compiler_params=pltpu.CompilerParams(
            dimension_semantics=("arbitrary", "arbitrary")),
    )(idx, time_table.T, W_time, weekday_table, W_weekday, node_table.T,
      W_node)
    return jnp.transpose(out, (2, 0, 1, 3))


# speedup vs baseline: 1.8285x; 1.2645x over previous
import jax
import jax.numpy as jnp
from jax import lax
from jax.experimental import pallas as pl
from jax.experimental.pallas import tpu as pltpu


def _dot0(a, b):
    return lax.dot_general(a, b, (((0,), (0,)), ((), ())),
                           preferred_element_type=jnp.float32)


def _body(idx_ref, ttT_ref, wt_ref, wkt_ref, wwk_ref, ntT_ref, wn_ref,
          out_ref, s_scr):
    t = pl.program_id(0)
    T = pl.num_programs(0)
    N = out_ref.shape[1]
    B = out_ref.shape[2]
    Vt = ttT_ref.shape[1]

    tproj = _dot0(ttT_ref[:], wt_ref[:])
    wproj = jnp.dot(wkt_ref[:], wwk_ref[:],
                    preferred_element_type=jnp.float32)
    wproj8 = jnp.concatenate(
        [wproj, jnp.zeros((1, wproj.shape[1]), jnp.float32)], axis=0)
    s_scr[:] = _dot0(ntT_ref[:], wn_ref[:])

    ti = idx_ref[t]
    wi = idx_ref[T + t]
    oh_t = (ti[None, :] == lax.broadcasted_iota(jnp.int32, (Vt, B), 0)
            ).astype(jnp.float32)
    oh_w = (wi[None, :] == lax.broadcasted_iota(jnp.int32, (8, B), 0)
            ).astype(jnp.float32)
    tv = _dot0(oh_t, tproj) + _dot0(oh_w, wproj8)

    for n in range(N):
        out_ref[0, n] = tv + s_scr[n, :]


def kernel(time, weekday, time_table, W_time, weekday_table, W_weekday,
           node_table, W_node):
    B, T, _ = time.shape
    N, _ = node_table.shape
    D = W_node.shape[1]
    idx = jnp.concatenate(
        [time.reshape(B, T).T, weekday.reshape(B, T).T],
        axis=0).astype(jnp.int32)

    def full(shape):
        return pl.BlockSpec(shape, lambda t: (0,) * len(shape))

    out = pl.pallas_call(
        _body,
        grid=(T,),
        in_specs=[full(idx.shape),
                  full((time_table.shape[1], time_table.shape[0])),
                  full(W_time.shape), full(weekday_table.shape),
                  full(W_weekday.shape),
                  full((node_table.shape[1], node_table.shape[0])),
                  full(W_node.shape)],
        out_specs=pl.BlockSpec((1, N, B, D), lambda t: (t, 0, 0, 0)),
        out_shape=jax.ShapeDtypeStruct((T, N, B, D), jnp.float32),
        scratch_shapes=[pltpu.VMEM((N, D), jnp.float32)],
        compiler_params=pltpu.CompilerParams(
            dimension_semantics=("arbitrary",)),
    )(idx, time_table.T, W_time, weekday_table, W_weekday, node_table.T,
      W_node)
    return jnp.transpose(out, (2, 0, 1, 3))
